# baseline (speedup 1.0000x reference)
import functools

import jax
import jax.numpy as jnp
from jax import lax
from jax.experimental import pallas as pl
from jax.experimental.pallas import tpu as pltpu
from jax.experimental.pallas import tpu_sc as plsc

B, L = 16, 1024
N = B * L
H2 = 512
HIDDEN = 2 * H2
NUM_RES = 32
N_ATOM_MASK = 9
NUM_CORES = 2
NUM_SUBCORES = 16
NW = NUM_CORES * NUM_SUBCORES
RPW = N // NW
T = 64
M = RPW // T + 1
TM = 32
MM = RPW // TM + 1
LANES = 16
GPR = RPW // LANES
JV = H2 // LANES


def _body(x_hbm, bpe_hbm, mask_hbm, tok_hbm, bpe_emb_hbm, amask_hbm,
          out_hbm, idx_tok_v, idx_bpe_v, mask_v, cbpe_f, cidxu_f, cidxm_f,
          cidxu2, cidxm2, rows_v, mrep_v, amask_v, tok_v,
          sem_g, sem_w, sem_m, sem_t):
    wid = lax.axis_index("s") * NUM_CORES + lax.axis_index("c")
    base = pl.multiple_of(wid * RPW, RPW)

    pltpu.sync_copy(x_hbm.at[pl.ds(base, RPW)], idx_tok_v)
    pltpu.sync_copy(bpe_hbm.at[pl.ds(base, RPW)], idx_bpe_v)
    pltpu.sync_copy(mask_hbm.at[pl.ds(base, RPW)], mask_v)
    pltpu.sync_copy(tok_hbm, tok_v)

    def tok_writes(g, _):
        ivec = idx_tok_v[pl.ds(g * LANES, LANES)]
        r0 = (base + g * LANES) * 2
        for t16 in range(LANES):
            pltpu.async_copy(tok_v.at[pl.ds(ivec[t16], 1)],
                             out_hbm.at[pl.ds(r0 + 2 * t16, 1)], sem_t)
        return 0

    lax.fori_loop(0, GPR, tok_writes, 0)

    pltpu.sync_copy(amask_hbm, amask_v)
    for j in range(JV):
        acc = amask_v[0, pl.ds(j * LANES, LANES)]
        for r in range(1, N_ATOM_MASK):
            acc = acc + amask_v[r, pl.ds(j * LANES, LANES)]
        mrep_v[0, pl.ds(j * LANES, LANES)] = acc

    def mrep_fill(t, _):
        @pl.loop(0, JV, unroll=8)
        def _(j):
            mrep_v[t, pl.ds(j * LANES, LANES)] = (
                mrep_v[0, pl.ds(j * LANES, LANES)])
        return 0

    lax.fori_loop(1, TM, mrep_fill, 0)

    def compact(g, carry):
        cu, cm = carry
        bvec = idx_bpe_v[pl.ds(g * LANES, LANES)]
        mvec = mask_v[pl.ds(g * LANES, LANES)]
        rvec = (base + g * LANES + lax.iota(jnp.int32, LANES)) * 2 + 1
        keep = mvec == 0
        ki = jnp.where(keep, jnp.ones((LANES,), jnp.int32),
                       jnp.zeros((LANES,), jnp.int32))
        pu = plsc.cumsum(ki)
        pm = (lax.iota(jnp.int32, LANES) + 1) - pu
        posu = cu + pu - 1
        posm = cm + pm - 1
        plsc.store_scatter(cbpe_f, [posu], bvec, mask=keep)
        plsc.store_scatter(cidxu_f, [posu], rvec, mask=keep)
        plsc.store_scatter(cidxm_f, [posm], rvec,
                           mask=jnp.logical_not(keep))
        pc = pu[LANES - 1]
        return cu + pc, cm + (LANES - pc)

    cu, cm = lax.fori_loop(0, GPR, compact, (jnp.int32(0), jnp.int32(0)))

    v0 = cidxm_f[pl.ds(0, LANES)][0]
    zeros = jnp.zeros((LANES,), jnp.int32)
    v0vec = zeros + v0
    lane = lax.iota(jnp.int32, LANES)

    def tailfill(g, _):
        s = g * LANES + lane
        bv = jnp.where(s < cu, cbpe_f[pl.ds(g * LANES, LANES)], zeros)
        cbpe_f[pl.ds(g * LANES, LANES)] = bv
        uv = jnp.where(s < cu, cidxu_f[pl.ds(g * LANES, LANES)], v0vec)
        cidxu_f[pl.ds(g * LANES, LANES)] = uv
        mv = jnp.where(s < cm, cidxm_f[pl.ds(g * LANES, LANES)], v0vec)
        cidxm_f[pl.ds(g * LANES, LANES)] = mv
        return 0

    lax.fori_loop(0, (M * T) // LANES, tailfill, 0)

    def stage2d_u(g, _):
        j = g // (T // LANES)
        o = (g % (T // LANES)) * LANES
        cidxu2[j, pl.ds(o, LANES)] = cidxu_f[pl.ds(g * LANES, LANES)]
        return 0

    lax.fori_loop(0, (M * T) // LANES, stage2d_u, 0)

    def stage2d_m(g, _):
        j = g // (TM // LANES)
        o = (g % (TM // LANES)) * LANES
        cidxm2[j, pl.ds(o, LANES)] = cidxm_f[pl.ds(g * LANES, LANES)]
        return 0

    lax.fori_loop(0, (MM * TM) // LANES, stage2d_m, 0)

    nchu = lax.div(cu + (T - 1), jnp.int32(T))

    def issue_gather(c, b):
        pltpu.async_copy(bpe_emb_hbm.at[cbpe_f.at[pl.ds(c * T, T)]],
                         rows_v.at[b], sem_g[b])

    def wait_gather(b):
        pltpu.make_async_copy(bpe_emb_hbm.at[pl.ds(0, T)],
                              rows_v.at[b], sem_g[b]).wait()

    def wait_scatter(b):
        pltpu.make_async_copy(rows_v.at[b], out_hbm.at[pl.ds(0, T)],
                              sem_w[b]).wait()

    @pl.when(nchu > 0)
    def _():
        issue_gather(0, 0)

    def step(kk, _):
        for bb in range(2):
            c = 2 * kk + bb

            @pl.when(c + 1 < nchu)
            def _():
                @pl.when(c >= 1)
                def _():
                    wait_scatter(1 - bb)

                issue_gather(c + 1, 1 - bb)

            @pl.when(c < nchu)
            def _():
                wait_gather(bb)
                pltpu.async_copy(rows_v.at[bb], out_hbm.at[cidxu2.at[c]],
                                 sem_w[bb])
        return 0

    lax.fori_loop(0, lax.div(nchu + 1, jnp.int32(2)), step, 0)

    @pl.when(nchu >= 2)
    def _():
        wait_scatter(0)
        wait_scatter(1)

    @pl.when(nchu == 1)
    def _():
        wait_scatter(0)

    nchm = lax.div(cm + (TM - 1), jnp.int32(TM))

    def mstep(k, _):
        pltpu.async_copy(mrep_v, out_hbm.at[cidxm2.at[k]], sem_m)
        return 0

    lax.fori_loop(0, nchm, mstep, 0)

    def mdrain(k, _):
        pltpu.make_async_copy(mrep_v, out_hbm.at[pl.ds(0, TM)], sem_m).wait()
        return 0

    lax.fori_loop(0, nchm, mdrain, 0)

    for _ in range(RPW // T):
        pltpu.make_async_copy(rows_v.at[0], out_hbm.at[pl.ds(0, T)],
                              sem_t).wait()


def _mesh_kernel():
    mesh = plsc.VectorSubcoreMesh(core_axis_name="c", subcore_axis_name="s")
    return functools.partial(
        pl.kernel,
        mesh=mesh,
        compiler_params=pltpu.CompilerParams(needs_layout_passes=False),
        out_type=jax.ShapeDtypeStruct((2 * N, H2), jnp.float32),
        scratch_types=[
            pltpu.VMEM((RPW,), jnp.int32),
            pltpu.VMEM((RPW,), jnp.int32),
            pltpu.VMEM((RPW,), jnp.int32),
            pltpu.VMEM((M * T,), jnp.int32),
            pltpu.VMEM((M * T,), jnp.int32),
            pltpu.VMEM((M * T,), jnp.int32),
            pltpu.VMEM((M, T), jnp.int32),
            pltpu.VMEM((MM, TM), jnp.int32),
            pltpu.VMEM((2, T, H2), jnp.float32),
            pltpu.VMEM((TM, H2), jnp.float32),
            pltpu.VMEM((N_ATOM_MASK, H2), jnp.float32),
            pltpu.VMEM((NUM_RES, H2), jnp.float32),
            [pltpu.SemaphoreType.DMA, pltpu.SemaphoreType.DMA],
            [pltpu.SemaphoreType.DMA, pltpu.SemaphoreType.DMA],
            pltpu.SemaphoreType.DMA,
            pltpu.SemaphoreType.DMA,
        ],
    )(_body)


@jax.jit
def kernel(x, bpe, mask_aa, token_embed, bpe_embed, atom_mask_embedding):
    out = _mesh_kernel()(x.reshape(N), bpe.reshape(N), mask_aa.reshape(N),
                         token_embed, bpe_embed, atom_mask_embedding)
    return out.reshape(B, L, HIDDEN)


# speedup vs baseline: 2.3280x; 2.3280x over previous
import functools

import jax
import jax.numpy as jnp
from jax import lax
from jax.experimental import pallas as pl
from jax.experimental.pallas import tpu as pltpu
from jax.experimental.pallas import tpu_sc as plsc

B, L = 16, 1024
N = B * L
H2 = 512
HIDDEN = 2 * H2
NUM_RES = 32
N_ATOM_MASK = 9
NUM_CORES = 2
NUM_SUBCORES = 16
NW = NUM_CORES * NUM_SUBCORES
RPW = N // NW
T = 64
NCHUNK = RPW // T
LANES = 16
JV = H2 // LANES


def _body(x_hbm, bpe_hbm, mask_hbm, tok_hbm, bpe_emb_hbm, amask_hbm,
          out_hbm, idx_tok_v, idx_bpe_v, mask_v, rows_bpe_v,
          amask_v, maskrow_v, tok_v, sem_g, sem_w):
    wid = lax.axis_index("s") * NUM_CORES + lax.axis_index("c")
    base = pl.multiple_of(wid * RPW, RPW)

    pltpu.sync_copy(x_hbm.at[pl.ds(base, RPW)], idx_tok_v)
    pltpu.sync_copy(bpe_hbm.at[pl.ds(base, RPW)], idx_bpe_v)
    pltpu.sync_copy(mask_hbm.at[pl.ds(base, RPW)], mask_v)
    pltpu.sync_copy(tok_hbm, tok_v)

    pltpu.sync_copy(amask_hbm, amask_v)
    for j in range(JV):
        acc = amask_v[0, pl.ds(j * LANES, LANES)]
        for r in range(1, N_ATOM_MASK):
            acc = acc + amask_v[r, pl.ds(j * LANES, LANES)]
        maskrow_v[pl.ds(j * LANES, LANES)] = acc

    def issue_gather(c, b):
        off = pl.multiple_of(c * T, T)
        pltpu.async_copy(bpe_emb_hbm.at[idx_bpe_v.at[pl.ds(off, T)]],
                         rows_bpe_v.at[b], sem_g[b])

    def wait_gather(b):
        pltpu.make_async_copy(bpe_emb_hbm.at[pl.ds(0, T)],
                              rows_bpe_v.at[b], sem_g[b]).wait()

    def issue_writes(c, b):
        off = pl.multiple_of(c * T, T)
        row0 = pl.multiple_of(base + off, T)
        pltpu.async_copy(rows_bpe_v.at[b],
                         out_hbm.at[pl.ds(row0, T), pl.ds(H2, H2)], sem_w[b])
        def tok_writes(g, _):
            ivec = idx_tok_v[pl.ds(off + g * LANES, LANES)]
            r0 = row0 + g * LANES
            for t16 in range(LANES):
                pltpu.async_copy(
                    tok_v.at[pl.ds(ivec[t16], 1)],
                    out_hbm.at[pl.ds(r0 + t16, 1), pl.ds(0, H2)], sem_w[b])
            return 0

        lax.fori_loop(0, T // LANES, tok_writes, 0)

    def wait_writes(b):
        pltpu.make_async_copy(rows_bpe_v.at[b],
                              out_hbm.at[pl.ds(base, T), pl.ds(H2, H2)],
                              sem_w[b]).wait()
        pltpu.make_async_copy(rows_bpe_v.at[b],
                              out_hbm.at[pl.ds(base, T), pl.ds(0, H2)],
                              sem_w[b]).wait()

    def fix_mask(c, b):
        off = pl.multiple_of(c * T, T)

        def fix(g, _):
            mvec = mask_v[pl.ds(off + g * LANES, LANES)]
            t0 = g * LANES
            for t16 in range(LANES):
                @pl.when(mvec[t16] != 0)
                def _():
                    @pl.loop(0, JV, unroll=8)
                    def _(j):
                        rows_bpe_v[b, t0 + t16, pl.ds(j * LANES, LANES)] = (
                            maskrow_v[pl.ds(j * LANES, LANES)])
            return 0

        lax.fori_loop(0, T // LANES, fix, 0)

    issue_gather(0, 0)

    def step(i, _):
        for bb in range(2):
            c = 2 * i + bb

            @pl.when(c + 1 < NCHUNK)
            def _():
                @pl.when(c >= 1)
                def _():
                    wait_writes(1 - bb)

                issue_gather(c + 1, 1 - bb)

            wait_gather(bb)
            fix_mask(c, bb)
            issue_writes(c, bb)
        return 0

    lax.fori_loop(0, NCHUNK // 2, step, 0, unroll=False)
    wait_writes(0)
    wait_writes(1)


def _mesh_kernel():
    mesh = plsc.VectorSubcoreMesh(core_axis_name="c", subcore_axis_name="s")
    return functools.partial(
        pl.kernel,
        mesh=mesh,
        out_type=jax.ShapeDtypeStruct((N, HIDDEN), jnp.float32),
        scratch_types=[
            pltpu.VMEM((RPW,), jnp.int32),
            pltpu.VMEM((RPW,), jnp.int32),
            pltpu.VMEM((RPW,), jnp.int32),
            pltpu.VMEM((2, T, H2), jnp.float32),
            pltpu.VMEM((N_ATOM_MASK, H2), jnp.float32),
            pltpu.VMEM((H2,), jnp.float32),
            pltpu.VMEM((NUM_RES, H2), jnp.float32),
            [pltpu.SemaphoreType.DMA, pltpu.SemaphoreType.DMA],
            [pltpu.SemaphoreType.DMA, pltpu.SemaphoreType.DMA],
        ],
    )(_body)


@jax.jit
def kernel(x, bpe, mask_aa, token_embed, bpe_embed, atom_mask_embedding):
    out = _mesh_kernel()(x.reshape(N), bpe.reshape(N), mask_aa.reshape(N),
                         token_embed, bpe_embed, atom_mask_embedding)
    return out.reshape(B, L, HIDDEN)
